# Initial kernel scaffold; baseline (speedup 1.0000x reference)
import jax
import jax.numpy as jnp
from jax import lax
from jax.experimental import pallas as pl
from jax.experimental.pallas import tpu as pltpu
from jax.experimental.pallas import tpu_sc as plsc

N = 100000
D = 128
G = 256
NC = 2
NS = 16
NW = NC * NS

CHUNK = 3136
LAST = N - (NW - 1) * CHUNK
BLK = 224
NBLK_FULL = CHUNK // BLK
NBLK_LAST = LAST // BLK
TAIL_LAST = LAST - NBLK_LAST * BLK

NEG_INF = float("-inf")


def _seg_partials(feat_hbm, ids_hbm, part_hbm, idsv, buf0, buf1, acc, sem0, sem1):
    c = lax.axis_index("c")
    s = lax.axis_index("s")
    w = s * NC + c
    base = w * CHUNK
    is_last = w == NW - 1

    neg = jnp.full((16,), NEG_INF, jnp.float32)

    def init_body(g, carry):
        for f in range(D // 16):
            acc[g, pl.ds(f * 16, 16)] = neg
        return carry

    lax.fori_loop(0, G, init_body, jnp.int32(0))

    def process_block(bufref, ids_off, rows):
        def grp(tt, carry):
            r0 = tt * 16
            for j in range(16):
                g = idsv[ids_off + r0 + j]
                for f in range(D // 16):
                    x = bufref[r0 + j, pl.ds(f * 16, 16)]
                    a = acc[g, pl.ds(f * 16, 16)]
                    acc[g, pl.ds(f * 16, 16)] = jnp.maximum(a, x)
            return carry

        lax.fori_loop(0, rows // 16, grp, jnp.int32(0))

    def run(nblk, tail):
        nrows = nblk * BLK + tail
        pltpu.sync_copy(ids_hbm.at[pl.ds(base, nrows)], idsv.at[pl.ds(0, nrows)])

        pltpu.async_copy(feat_hbm.at[pl.ds(base, BLK), :], buf0, sem0)

        npair = nblk // 2

        def pair_body(t, carry):
            b0 = 2 * t
            pltpu.make_async_copy(feat_hbm.at[pl.ds(base, BLK), :], buf0, sem0).wait()
            pltpu.async_copy(
                feat_hbm.at[pl.ds(base + (b0 + 1) * BLK, BLK), :], buf1, sem1
            )
            process_block(buf0, b0 * BLK, BLK)
            pltpu.make_async_copy(feat_hbm.at[pl.ds(base, BLK), :], buf1, sem1).wait()

            @pl.when(b0 + 2 < nblk)
            def _():
                pltpu.async_copy(
                    feat_hbm.at[pl.ds(base + (b0 + 2) * BLK, BLK), :], buf0, sem0
                )

            process_block(buf1, (b0 + 1) * BLK, BLK)
            return carry

        lax.fori_loop(0, npair, pair_body, jnp.int32(0))

        if tail:
            pltpu.sync_copy(
                feat_hbm.at[pl.ds(base + nblk * BLK, tail), :],
                buf0.at[pl.ds(0, tail), :],
            )
            process_block(buf0, nblk * BLK, tail)

    @pl.when(jnp.logical_not(is_last))
    def _():
        run(NBLK_FULL, 0)

    @pl.when(is_last)
    def _():
        run(NBLK_LAST, TAIL_LAST)

    pltpu.sync_copy(acc, part_hbm.at[w])


_mesh = plsc.VectorSubcoreMesh(
    core_axis_name="c", subcore_axis_name="s", num_cores=NC, num_subcores=NS
)

_phase1 = pl.kernel(
    _seg_partials,
    out_type=jax.ShapeDtypeStruct((NW, G, D), jnp.float32),
    mesh=_mesh,
    scratch_types=[
        pltpu.VMEM((CHUNK,), jnp.int32),
        pltpu.VMEM((BLK, D), jnp.float32),
        pltpu.VMEM((BLK, D), jnp.float32),
        pltpu.VMEM((G, D), jnp.float32),
        pltpu.SemaphoreType.DMA,
        pltpu.SemaphoreType.DMA,
    ],
)


def _combine_body(parts_ref, out_ref):
    out_ref[...] = jnp.max(parts_ref[...], axis=0)


_combine = pl.pallas_call(
    _combine_body,
    out_shape=jax.ShapeDtypeStruct((G, D), jnp.float32),
)


@jax.jit
def _impl(feat, ids):
    partials = _phase1(feat, ids)
    return _combine(partials)


def kernel(feat0, segment_ids):
    return _impl(feat0[..., 0], segment_ids)


# speedup vs baseline: 2.7501x; 2.7501x over previous
import jax
import jax.numpy as jnp
from jax import lax
from jax.experimental import pallas as pl
from jax.experimental.pallas import tpu as pltpu
from jax.experimental.pallas import tpu_sc as plsc

N = 100000
D = 128
G = 256
NC = 2
NS = 16
NW = NC * NS

CHUNK = 3136
LAST = N - (NW - 1) * CHUNK
BLK = 224
NBLK_FULL = CHUNK // BLK
NBLK_LAST = LAST // BLK
TAIL_LAST = LAST - NBLK_LAST * BLK

NEG_INF = float("-inf")


def _seg_partials(feat_hbm, ids_hbm, part_hbm, idsv, buf0, buf1, acc, sem0, sem1):
    c = lax.axis_index("c")
    s = lax.axis_index("s")
    w = s * NC + c
    base = w * CHUNK
    is_last = w == NW - 1

    neg = jnp.full((16,), NEG_INF, jnp.float32)

    def init_body(g, carry):
        for f in range(D // 16):
            acc[g, pl.ds(f * 16, 16)] = neg
        return carry

    lax.fori_loop(0, G, init_body, jnp.int32(0))

    def process_block(bufref, ids_off, rows):
        def grp(tt, carry):
            r0 = tt * 16
            idvec = idsv[pl.ds(ids_off + r0, 16)]
            for j in range(16):
                g = idvec[j]
                for f in range(D // 16):
                    x = bufref[r0 + j, pl.ds(f * 16, 16)]
                    a = acc[g, pl.ds(f * 16, 16)]
                    acc[g, pl.ds(f * 16, 16)] = jnp.maximum(a, x)
            return carry

        lax.fori_loop(0, rows // 16, grp, jnp.int32(0))

    def run(nblk, tail):
        nrows = nblk * BLK + tail
        pltpu.sync_copy(ids_hbm.at[pl.ds(base, nrows)], idsv.at[pl.ds(0, nrows)])

        pltpu.async_copy(feat_hbm.at[pl.ds(base, BLK), :], buf0, sem0)

        npair = nblk // 2

        def pair_body(t, carry):
            b0 = 2 * t
            pltpu.make_async_copy(feat_hbm.at[pl.ds(base, BLK), :], buf0, sem0).wait()
            pltpu.async_copy(
                feat_hbm.at[pl.ds(base + (b0 + 1) * BLK, BLK), :], buf1, sem1
            )
            process_block(buf0, b0 * BLK, BLK)
            pltpu.make_async_copy(feat_hbm.at[pl.ds(base, BLK), :], buf1, sem1).wait()

            @pl.when(b0 + 2 < nblk)
            def _():
                pltpu.async_copy(
                    feat_hbm.at[pl.ds(base + (b0 + 2) * BLK, BLK), :], buf0, sem0
                )

            process_block(buf1, (b0 + 1) * BLK, BLK)
            return carry

        lax.fori_loop(0, npair, pair_body, jnp.int32(0))

        if tail:
            pltpu.sync_copy(
                feat_hbm.at[pl.ds(base + nblk * BLK, tail), :],
                buf0.at[pl.ds(0, tail), :],
            )
            process_block(buf0, nblk * BLK, tail)

    @pl.when(jnp.logical_not(is_last))
    def _():
        run(NBLK_FULL, 0)

    @pl.when(is_last)
    def _():
        run(NBLK_LAST, TAIL_LAST)

    pltpu.sync_copy(acc, part_hbm.at[w])


_mesh = plsc.VectorSubcoreMesh(
    core_axis_name="c", subcore_axis_name="s", num_cores=NC, num_subcores=NS
)

_phase1 = pl.kernel(
    _seg_partials,
    out_type=jax.ShapeDtypeStruct((NW, G, D), jnp.float32),
    mesh=_mesh,
    scratch_types=[
        pltpu.VMEM((CHUNK,), jnp.int32),
        pltpu.VMEM((BLK, D), jnp.float32),
        pltpu.VMEM((BLK, D), jnp.float32),
        pltpu.VMEM((G, D), jnp.float32),
        pltpu.SemaphoreType.DMA,
        pltpu.SemaphoreType.DMA,
    ],
)


def _combine_body(parts_ref, out_ref):
    out_ref[...] = jnp.max(parts_ref[...], axis=0)


_combine = pl.pallas_call(
    _combine_body,
    out_shape=jax.ShapeDtypeStruct((G, D), jnp.float32),
)


@jax.jit
def _impl(feat, ids):
    partials = _phase1(feat, ids)
    return _combine(partials)


def kernel(feat0, segment_ids):
    return _impl(feat0[..., 0], segment_ids)
